# baseline (speedup 1.0000x reference)
import functools

import jax
import jax.numpy as jnp
from jax import lax
from jax.experimental import pallas as pl
from jax.experimental.pallas import tpu as pltpu
from jax.experimental.pallas import tpu_sc as plsc

_B, _S, _D = 4, 8192, 1024
_NC, _NS = 2, 16
_NW = _NC * _NS
_SPW = _S // _NW
_SUB = 32
_NSUB = _SPW // _SUB
_NV = _D // 16
_LANES = 16

_mesh = plsc.VectorSubcoreMesh(core_axis_name="c", subcore_axis_name="s")


@functools.partial(
    pl.kernel,
    out_type=jax.ShapeDtypeStruct((_B * _S, _D), jnp.float32),
    mesh=_mesh,
    scratch_types=[
        pltpu.VMEM((_SUB, _D), jnp.float32),
        pltpu.VMEM((_SUB, _D), jnp.float32),
    ],
)
def _pos_add(x_hbm, pos_hbm, out_hbm, pos_v, buf_v):
    wid = lax.axis_index("s") * _NC + lax.axis_index("c")
    s0 = wid * _SPW

    def sub(j, carry):
        s_base = s0 + j * _SUB
        pltpu.sync_copy(pos_hbm.at[pl.ds(s_base, _SUB)], pos_v)
        for b in range(_B):
            row = b * _S + s_base
            pltpu.sync_copy(x_hbm.at[pl.ds(row, _SUB)], buf_v)

            def rbody(r, c2):
                for k in range(_NV):
                    plsc.addupdate(
                        buf_v.at[r, pl.ds(k * _LANES, _LANES)],
                        pos_v[r, pl.ds(k * _LANES, _LANES)],
                    )
                return c2

            lax.fori_loop(0, _SUB, rbody, 0)
            pltpu.sync_copy(buf_v, out_hbm.at[pl.ds(row, _SUB)])
        return carry

    lax.fori_loop(0, _NSUB, sub, 0)


def kernel(x, pos_table):
    out = _pos_add(x.reshape(_B * _S, _D), pos_table)
    return out.reshape(_B, _S, _D)


# speedup vs baseline: 1.6084x; 1.6084x over previous
import functools

import jax
import jax.numpy as jnp
from jax import lax
from jax.experimental import pallas as pl
from jax.experimental.pallas import tpu as pltpu
from jax.experimental.pallas import tpu_sc as plsc

_B, _S, _D = 4, 8192, 1024
_NC, _NS = 2, 16
_NW = _NC * _NS
_SPW = _S // _NW
_SUB = 32
_NSUB = _SPW // _SUB
_NV = _D // 16
_LANES = 16

_mesh = plsc.VectorSubcoreMesh(core_axis_name="c", subcore_axis_name="s")


@functools.partial(
    pl.kernel,
    out_type=jax.ShapeDtypeStruct((_B * _S, _D), jnp.float32),
    mesh=_mesh,
    scratch_types=[
        pltpu.VMEM((_SUB, _D), jnp.float32),
        pltpu.VMEM((_SUB, _D), jnp.float32),
    ],
)
def _pos_add(x_hbm, pos_hbm, out_hbm, pos_v, buf_v):
    wid = lax.axis_index("s") * _NC + lax.axis_index("c")
    s0 = wid * _SPW

    def sub(j, carry):
        s_base = s0 + j * _SUB
        pltpu.sync_copy(pos_hbm.at[pl.ds(s_base, _SUB)], pos_v)
        for b in range(_B):
            row = b * _S + s_base
            pltpu.sync_copy(x_hbm.at[pl.ds(row, _SUB)], buf_v)

            @plsc.parallel_loop(0, _SUB, 1)
            def _add(r):
                for k in range(_NV):
                    plsc.addupdate(
                        buf_v.at[r, pl.ds(k * _LANES, _LANES)],
                        pos_v[r, pl.ds(k * _LANES, _LANES)],
                    )

            pltpu.sync_copy(buf_v, out_hbm.at[pl.ds(row, _SUB)])
        return carry

    lax.fori_loop(0, _NSUB, sub, 0)


def kernel(x, pos_table):
    out = _pos_add(x.reshape(_B * _S, _D), pos_table)
    return out.reshape(_B, _S, _D)


# speedup vs baseline: 2.3620x; 1.4685x over previous
import functools

import jax
import jax.numpy as jnp
from jax import lax
from jax.experimental import pallas as pl
from jax.experimental.pallas import tpu as pltpu
from jax.experimental.pallas import tpu_sc as plsc

_B, _S, _D = 4, 8192, 1024
_NC, _NS = 2, 16
_NW = _NC * _NS
_SPW = _S // _NW
_SUB = 16
_NSUB = _SPW // _SUB
_NSTEP = _NSUB * _B
_NV = _D // 16
_LANES = 16

_mesh = plsc.VectorSubcoreMesh(core_axis_name="c", subcore_axis_name="s")


@functools.partial(
    pl.kernel,
    out_type=jax.ShapeDtypeStruct((_B * _S, _D), jnp.float32),
    mesh=_mesh,
    scratch_types=[
        *([pltpu.VMEM((_SUB, _D), jnp.float32)] * 4),
        *([pltpu.VMEM((_SUB, _D), jnp.float32)] * 2),
        *([pltpu.SemaphoreType.DMA] * 10),
    ],
)
def _pos_add(x_hbm, pos_hbm, out_hbm,
             xb0, xb1, xb2, xb3, pb0, pb1,
             sx0, sx1, sx2, sx3, so0, so1, so2, so3, sp0, sp1):
    xbufs = (xb0, xb1, xb2, xb3)
    semx = (sx0, sx1, sx2, sx3)
    semo = (so0, so1, so2, so3)
    pbufs = (pb0, pb1)
    semp = (sp0, sp1)

    wid = lax.axis_index("s") * _NC + lax.axis_index("c")
    s0 = wid * _SPW

    def row_of(jj, b):
        return b * _S + s0 + jj * _SUB

    def start_x(jj, b, p):
        pltpu.async_copy(x_hbm.at[pl.ds(row_of(jj, b), _SUB)],
                         xbufs[p], semx[p])

    def wait_x(p):
        pltpu.make_async_copy(x_hbm.at[pl.ds(0, _SUB)],
                              xbufs[p], semx[p]).wait()

    def start_p(jj, p):
        pltpu.async_copy(pos_hbm.at[pl.ds(s0 + jj * _SUB, _SUB)],
                         pbufs[p], semp[p])

    def wait_p(p):
        pltpu.make_async_copy(pos_hbm.at[pl.ds(0, _SUB)],
                              pbufs[p], semp[p]).wait()

    def start_o(jj, b, p):
        pltpu.async_copy(xbufs[p], out_hbm.at[pl.ds(row_of(jj, b), _SUB)],
                         semo[p])

    def wait_o(p):
        pltpu.make_async_copy(xbufs[p], out_hbm.at[pl.ds(0, _SUB)],
                              semo[p]).wait()

    start_p(0, 0)
    start_p(1, 1)
    start_x(0, 0, 0)
    start_x(0, 1, 1)

    def body(g, carry):
        for t in range(8):
            jj = g + (t >> 2)
            b = t & 3
            p = t & 3
            pp = (t >> 2) & 1
            p2 = (t + 2) & 3
            s2 = 4 * g + t + 2

            if t >= 2:
                wait_o(p2)
            else:
                @pl.when(g > 0)
                def _():
                    wait_o(p2)

            @pl.when(s2 < _NSTEP)
            def _():
                jj2 = g + ((t + 2) >> 2)
                start_x(jj2, (t + 2) & 3, p2)

            if t % 4 == 0:
                wait_p(pp)
            wait_x(p)

            @plsc.parallel_loop(0, _SUB, 1)
            def _add(r):
                for k in range(_NV):
                    plsc.addupdate(
                        xbufs[p].at[r, pl.ds(k * _LANES, _LANES)],
                        pbufs[pp][r, pl.ds(k * _LANES, _LANES)],
                    )

            start_o(jj, b, p)

            if t % 4 == 3:
                @pl.when(jj + 2 < _NSUB)
                def _():
                    start_p(jj + 2, pp)
        return carry

    pl.loop(0, _NSUB, step=2)(lambda g: body(g, None))

    wait_o(2)
    wait_o(3)


def kernel(x, pos_table):
    out = _pos_add(x.reshape(_B * _S, _D), pos_table)
    return out.reshape(_B, _S, _D)


# speedup vs baseline: 2.8467x; 1.2052x over previous
import functools

import jax
import jax.numpy as jnp
from jax import lax
from jax.experimental import pallas as pl
from jax.experimental.pallas import tpu as pltpu
from jax.experimental.pallas import tpu_sc as plsc

_B, _S, _D = 4, 8192, 1024
_NC, _NS = 2, 16
_NW = _NC * _NS
_SPW = _S // _NW
_SUB = 16
_NSUB = _SPW // _SUB
_NSTEP = _NSUB * _B
_NV = _D // 16
_LANES = 16

_mesh = plsc.VectorSubcoreMesh(core_axis_name="c", subcore_axis_name="s")


@functools.partial(
    pl.kernel,
    out_type=jax.ShapeDtypeStruct((_B * _S, _D), jnp.float32),
    mesh=_mesh,
    scratch_types=[
        *([pltpu.VMEM((_SUB, _D), jnp.float32)] * 4),
        *([pltpu.VMEM((_SUB, _D), jnp.float32)] * 2),
        *([pltpu.SemaphoreType.DMA] * 10),
    ],
)
def _pos_add(x_hbm, pos_hbm, out_hbm,
             xb0, xb1, xb2, xb3, pb0, pb1,
             sx0, sx1, sx2, sx3, so0, so1, so2, so3, sp0, sp1):
    xbufs = (xb0, xb1, xb2, xb3)
    semx = (sx0, sx1, sx2, sx3)
    semo = (so0, so1, so2, so3)
    pbufs = (pb0, pb1)
    semp = (sp0, sp1)

    wid = lax.axis_index("s") * _NC + lax.axis_index("c")
    s0 = wid * _SPW

    def row_of(jj, b):
        return b * _S + s0 + jj * _SUB

    def start_x(jj, b, p):
        pltpu.async_copy(x_hbm.at[pl.ds(row_of(jj, b), _SUB)],
                         xbufs[p], semx[p])

    def wait_x(p):
        pltpu.make_async_copy(x_hbm.at[pl.ds(0, _SUB)],
                              xbufs[p], semx[p]).wait()

    def start_p(jj, p):
        pltpu.async_copy(pos_hbm.at[pl.ds(s0 + jj * _SUB, _SUB)],
                         pbufs[p], semp[p])

    def wait_p(p):
        pltpu.make_async_copy(pos_hbm.at[pl.ds(0, _SUB)],
                              pbufs[p], semp[p]).wait()

    def start_o(jj, b, p):
        pltpu.async_copy(xbufs[p], out_hbm.at[pl.ds(row_of(jj, b), _SUB)],
                         semo[p])

    def wait_o(p):
        pltpu.make_async_copy(xbufs[p], out_hbm.at[pl.ds(0, _SUB)],
                              semo[p]).wait()

    start_p(0, 0)
    start_p(1, 1)
    start_x(0, 0, 0)
    start_x(0, 1, 1)

    def body(g, carry):
        for t in range(8):
            jj = g + (t >> 2)
            b = t & 3
            p = t & 3
            pp = (t >> 2) & 1
            p2 = (t + 2) & 3
            s2 = 4 * g + t + 2

            if t >= 2:
                wait_o(p2)
            else:
                @pl.when(g > 0)
                def _():
                    wait_o(p2)

            @pl.when(s2 < _NSTEP)
            def _():
                jj2 = g + ((t + 2) >> 2)
                start_x(jj2, (t + 2) & 3, p2)

            if t % 4 == 0:
                wait_p(pp)
            wait_x(p)

            @plsc.parallel_loop(0, _SUB, 1)
            def _add(r):
                for k0 in range(0, _NV, 8):
                    vs = [pbufs[pp][r, pl.ds((k0 + k) * _LANES, _LANES)]
                          for k in range(8)]
                    for k in range(8):
                        plsc.addupdate(
                            xbufs[p].at[r, pl.ds((k0 + k) * _LANES, _LANES)],
                            vs[k],
                        )

            start_o(jj, b, p)

            if t % 4 == 3:
                @pl.when(jj + 2 < _NSUB)
                def _():
                    start_p(jj + 2, pp)
        return carry

    pl.loop(0, _NSUB, step=2)(lambda g: body(g, None))

    wait_o(2)
    wait_o(3)


def kernel(x, pos_table):
    out = _pos_add(x.reshape(_B * _S, _D), pos_table)
    return out.reshape(_B, _S, _D)


# speedup vs baseline: 2.9216x; 1.0263x over previous
import functools

import jax
import jax.numpy as jnp
from jax import lax
from jax.experimental import pallas as pl
from jax.experimental.pallas import tpu as pltpu
from jax.experimental.pallas import tpu_sc as plsc

_B, _S, _D = 4, 8192, 1024
_NC, _NS = 2, 16
_NW = _NC * _NS
_SPW = _S // _NW
_SUB = 16
_NSUB = _SPW // _SUB
_NSTEP = _NSUB * _B
_NV = _D // 16
_LANES = 16

_mesh = plsc.VectorSubcoreMesh(core_axis_name="c", subcore_axis_name="s")


@functools.partial(
    pl.kernel,
    out_type=jax.ShapeDtypeStruct((_B * _S, _D), jnp.float32),
    mesh=_mesh,
    scratch_types=[
        *([pltpu.VMEM((_SUB, _D), jnp.float32)] * 4),
        *([pltpu.VMEM((_SUB, _D), jnp.float32)] * 2),
        *([pltpu.SemaphoreType.DMA] * 10),
    ],
)
def _pos_add(x_hbm, pos_hbm, out_hbm,
             xb0, xb1, xb2, xb3, pb0, pb1,
             sx0, sx1, sx2, sx3, so0, so1, so2, so3, sp0, sp1):
    xbufs = (xb0, xb1, xb2, xb3)
    semx = (sx0, sx1, sx2, sx3)
    semo = (so0, so1, so2, so3)
    pbufs = (pb0, pb1)
    semp = (sp0, sp1)

    wid = lax.axis_index("s") * _NC + lax.axis_index("c")
    s0 = wid * _SPW

    def row_of(jj, b):
        return b * _S + s0 + jj * _SUB

    def start_x(jj, b, p):
        pltpu.async_copy(x_hbm.at[pl.ds(row_of(jj, b), _SUB)],
                         xbufs[p], semx[p])

    def wait_x(p):
        pltpu.make_async_copy(x_hbm.at[pl.ds(0, _SUB)],
                              xbufs[p], semx[p]).wait()

    def start_p(jj, p):
        pltpu.async_copy(pos_hbm.at[pl.ds(s0 + jj * _SUB, _SUB)],
                         pbufs[p], semp[p])

    def wait_p(p):
        pltpu.make_async_copy(pos_hbm.at[pl.ds(0, _SUB)],
                              pbufs[p], semp[p]).wait()

    def start_o(jj, b, p):
        pltpu.async_copy(xbufs[p], out_hbm.at[pl.ds(row_of(jj, b), _SUB)],
                         semo[p])

    def wait_o(p):
        pltpu.make_async_copy(xbufs[p], out_hbm.at[pl.ds(0, _SUB)],
                              semo[p]).wait()

    start_p(0, 0)
    start_p(1, 1)
    start_x(0, 0, 0)
    start_x(0, 1, 1)

    def body(g, carry):
        for t in range(8):
            jj = g + (t >> 2)
            b = t & 3
            p = t & 3
            pp = (t >> 2) & 1
            p2 = (t + 2) & 3
            s2 = 4 * g + t + 2

            if t >= 2:
                wait_o(p2)
            else:
                @pl.when(g > 0)
                def _():
                    wait_o(p2)

            @pl.when(s2 < _NSTEP)
            def _():
                jj2 = g + ((t + 2) >> 2)
                start_x(jj2, (t + 2) & 3, p2)

            if t % 4 == 0:
                wait_p(pp)
            wait_x(p)

            @plsc.parallel_loop(0, _SUB, 1)
            def _add(r):
                def loads(k0):
                    return [pbufs[pp][r, pl.ds((k0 + k) * _LANES, _LANES)]
                            for k in range(8)]

                def stores(k0, vs):
                    for k in range(8):
                        plsc.addupdate(
                            xbufs[p].at[r, pl.ds((k0 + k) * _LANES, _LANES)],
                            vs[k],
                        )

                vs = loads(0)
                for k0 in range(8, _NV, 8):
                    nxt = loads(k0)
                    stores(k0 - 8, vs)
                    vs = nxt
                stores(_NV - 8, vs)

            start_o(jj, b, p)

            if t % 4 == 3:
                @pl.when(jj + 2 < _NSUB)
                def _():
                    start_p(jj + 2, pp)
        return carry

    pl.loop(0, _NSUB, step=2)(lambda g: body(g, None))

    wait_o(2)
    wait_o(3)


def kernel(x, pos_table):
    out = _pos_add(x.reshape(_B * _S, _D), pos_table)
    return out.reshape(_B, _S, _D)


# speedup vs baseline: 3.0399x; 1.0405x over previous
import functools

import jax
import jax.numpy as jnp
from jax import lax
from jax.experimental import pallas as pl
from jax.experimental.pallas import tpu as pltpu
from jax.experimental.pallas import tpu_sc as plsc

_B, _S, _D = 4, 8192, 1024
_NC, _NS = 2, 16
_NW = _NC * _NS
_SPW = _S // _NW
_SUB = 4
_NSTEP = _SPW // _SUB
_NV = _D // 16
_NVEC = _SUB * _NV
_LANES = 16

_mesh = plsc.VectorSubcoreMesh(core_axis_name="c", subcore_axis_name="s")


@functools.partial(
    pl.kernel,
    out_type=jax.ShapeDtypeStruct((_B, _S, _D), jnp.float32),
    mesh=_mesh,
    scratch_types=[
        *([pltpu.VMEM((_B, _SUB, _D), jnp.float32)] * 4),
        *([pltpu.VMEM((_SUB, _D), jnp.float32)] * 2),
        *([pltpu.SemaphoreType.DMA] * 10),
    ],
)
def _pos_add(x_hbm, pos_hbm, out_hbm,
             rb0, rb1, rb2, rb3, pb0, pb1,
             sx0, sx1, sx2, sx3, so0, so1, so2, so3,
             sp0, sp1):
    rbufs = (rb0, rb1, rb2, rb3)
    pbufs = (pb0, pb1)
    semx = (sx0, sx1, sx2, sx3)
    semo = (so0, so1, so2, so3)
    semp = (sp0, sp1)

    wid = lax.axis_index("s") * _NC + lax.axis_index("c")
    s0 = wid * _SPW

    def srow_of(s):
        return s0 + s * _SUB

    def start_x(s, p):
        pltpu.async_copy(x_hbm.at[:, pl.ds(srow_of(s), _SUB), :],
                         rbufs[p], semx[p])

    def wait_x(p):
        pltpu.make_async_copy(x_hbm.at[:, pl.ds(0, _SUB), :],
                              rbufs[p], semx[p]).wait()

    def start_p(s, q):
        pltpu.async_copy(pos_hbm.at[pl.ds(srow_of(s), _SUB)],
                         pbufs[q], semp[q])

    def wait_p(q):
        pltpu.make_async_copy(pos_hbm.at[pl.ds(0, _SUB)],
                              pbufs[q], semp[q]).wait()

    def start_o(s, p):
        pltpu.async_copy(rbufs[p],
                         out_hbm.at[:, pl.ds(srow_of(s), _SUB), :], semo[p])

    def wait_o(p):
        pltpu.make_async_copy(rbufs[p],
                              out_hbm.at[:, pl.ds(0, _SUB), :],
                              semo[p]).wait()

    start_p(0, 0)
    start_p(1, 1)
    start_x(0, 0)
    start_x(1, 1)

    def body(g):
        for t in range(4):
            s = g + t
            p = t
            p2 = (t + 2) & 3
            q = t & 1

            if t >= 2:
                wait_o(p2)
            else:
                @pl.when(g > 0)
                def _():
                    wait_o(p2)

            @pl.when(s + 2 < _NSTEP)
            def _():
                start_x(s + 2, p2)

            wait_p(q)
            wait_x(p)

            @plsc.parallel_loop(0, _NVEC, 1, unroll=8)
            def _add(i):
                r = i >> 6
                k = (i & (_NV - 1)) * _LANES
                v = pbufs[q][r, pl.ds(k, _LANES)]
                for b in range(_B):
                    plsc.addupdate(rbufs[p].at[b, r, pl.ds(k, _LANES)], v)

            start_o(s, p)

            @pl.when(s + 2 < _NSTEP)
            def _():
                start_p(s + 2, q)

    pl.loop(0, _NSTEP, step=4)(body)

    wait_o(2)
    wait_o(3)


def kernel(x, pos_table):
    return _pos_add(x, pos_table)
